# Initial kernel scaffold; baseline (speedup 1.0000x reference)
import jax
import jax.numpy as jnp
from jax.experimental import pallas as pl


def kernel(x, W1, b1, W2, b2):
    raise NotImplementedError("write your pallas kernel here")


# speedup vs baseline: 4.5530x; 4.5530x over previous
import jax
import jax.numpy as jnp
from jax.experimental import pallas as pl
from jax.experimental.pallas import tpu as pltpu

N_TOK = 32768
D_IN = 768
D_HID = 256
N_EXP = 64
TOP_K = 8
TILE = 512
NUM_TILES = N_TOK // TILE


def _gating_body(x_ref, w1_ref, b1_ref, w2_ref, b2_ref,
                 idx_ref, gate_ref, loss_ref, acc_ref):
    i = pl.program_id(0)

    @pl.when(i == 0)
    def _init():
        acc_ref[...] = jnp.zeros_like(acc_ref)
        loss_ref[...] = jnp.zeros((1, 1), jnp.float32)

    x = x_ref[...]
    h = jnp.maximum(
        jnp.dot(x, w1_ref[...], preferred_element_type=jnp.float32)
        + b1_ref[...], 0.0)
    logits = (jnp.dot(h, w2_ref[...], preferred_element_type=jnp.float32)
              + b2_ref[...])

    m = jnp.max(logits, axis=-1, keepdims=True)
    e = jnp.exp(logits - m)
    probs = e / jnp.sum(e, axis=-1, keepdims=True)
    probs_colsum = jnp.sum(probs, axis=0, keepdims=True)

    iota = jax.lax.broadcasted_iota(jnp.int32, logits.shape, 1)
    neg = jnp.float32(-jnp.finfo(jnp.float32).max)
    work = logits
    idx_cols = []
    val_cols = []
    cnt = jnp.zeros((1, N_EXP), jnp.float32)
    for _ in range(TOP_K):
        mk = jnp.max(work, axis=-1, keepdims=True)
        ik = jnp.min(jnp.where(work == mk, iota, N_EXP),
                     axis=-1, keepdims=True)
        onehot = iota == ik
        cnt = cnt + jnp.sum(onehot.astype(jnp.float32), axis=0, keepdims=True)
        work = jnp.where(onehot, neg, work)
        idx_cols.append(ik)
        val_cols.append(mk)

    idx_ref[...] = jnp.concatenate(idx_cols, axis=1)
    v = jnp.concatenate(val_cols, axis=1)
    g = jnp.exp(v - v[:, 0:1])
    gate_ref[...] = g / jnp.sum(g, axis=-1, keepdims=True)

    acc_ref[...] += jnp.concatenate([probs_colsum, cnt], axis=0)

    @pl.when(i == NUM_TILES - 1)
    def _finalize():
        a = acc_ref[...]
        loss_ref[...] = (N_EXP / (N_TOK * N_TOK)) * jnp.sum(
            a[0:1, :] * a[1:2, :], axis=-1, keepdims=True)


def kernel(x, W1, b1, W2, b2):
    b1r = b1.reshape(1, D_HID)
    b2r = b2.reshape(1, N_EXP)
    idx, gates, loss = pl.pallas_call(
        _gating_body,
        grid=(NUM_TILES,),
        in_specs=[
            pl.BlockSpec((TILE, D_IN), lambda i: (i, 0)),
            pl.BlockSpec((D_IN, D_HID), lambda i: (0, 0)),
            pl.BlockSpec((1, D_HID), lambda i: (0, 0)),
            pl.BlockSpec((D_HID, N_EXP), lambda i: (0, 0)),
            pl.BlockSpec((1, N_EXP), lambda i: (0, 0)),
        ],
        out_specs=[
            pl.BlockSpec((TILE, TOP_K), lambda i: (i, 0)),
            pl.BlockSpec((TILE, TOP_K), lambda i: (i, 0)),
            pl.BlockSpec((1, 1), lambda i: (0, 0)),
        ],
        out_shape=[
            jax.ShapeDtypeStruct((N_TOK, TOP_K), jnp.int32),
            jax.ShapeDtypeStruct((N_TOK, TOP_K), jnp.float32),
            jax.ShapeDtypeStruct((1, 1), jnp.float32),
        ],
        scratch_shapes=[pltpu.VMEM((2, N_EXP), jnp.float32)],
    )(x, W1, b1r, W2, b2r)
    return idx, gates, loss.reshape(())


# speedup vs baseline: 13.3810x; 2.9389x over previous
import jax
import jax.numpy as jnp
from jax.experimental import pallas as pl
from jax.experimental.pallas import tpu as pltpu

N_TOK = 32768
D_IN = 768
D_HID = 256
N_EXP = 64
TOP_K = 8
TILE = 512
NUM_TILES = N_TOK // TILE


def _gating_body(x_ref, w1_ref, b1_ref, w2_ref, b2_ref,
                 idx_ref, gate_ref, loss_ref, accp_ref, accc_ref):
    i = pl.program_id(0)

    @pl.when(i == 0)
    def _init():
        accp_ref[...] = jnp.zeros_like(accp_ref)
        accc_ref[...] = jnp.zeros_like(accc_ref)
        loss_ref[...] = jnp.zeros((1, 1), jnp.float32)

    x = x_ref[...]
    h = jnp.maximum(
        jnp.dot(x, w1_ref[...], preferred_element_type=jnp.float32)
        + b1_ref[...], 0.0)
    logits_t = jax.lax.dot_general(
        w2_ref[...], h, (((0,), (1,)), ((), ())),
        preferred_element_type=jnp.float32) + b2_ref[...]

    m = jnp.max(logits_t, axis=0, keepdims=True)
    e = jnp.exp(logits_t - m)
    s = jnp.sum(e, axis=0, keepdims=True)
    rs = 1.0 / s
    psum = jax.lax.dot_general(
        e, rs, (((1,), (1,)), ((), ())),
        preferred_element_type=jnp.float32)

    iota_e = jax.lax.broadcasted_iota(jnp.int32, logits_t.shape, 0)
    neg = jnp.float32(-jnp.finfo(jnp.float32).max)
    work = logits_t
    idx_rows = []
    val_rows = []
    for _ in range(TOP_K):
        mk = jnp.max(work, axis=0, keepdims=True)
        key = jnp.where(work == mk, iota_e, N_EXP)
        ik = jnp.min(key, axis=0, keepdims=True)
        work = jnp.where(iota_e == ik, neg, work)
        idx_rows.append(ik)
        val_rows.append(mk)

    idx_ref[...] = jnp.concatenate(idx_rows, axis=0)
    v = jnp.concatenate(val_rows, axis=0)
    g = jnp.exp(v - v[0:1, :])
    gate_ref[...] = g / jnp.sum(g, axis=0, keepdims=True)

    selmask = (work != logits_t).astype(jnp.float32)
    ones_t = jnp.ones((1, TILE), jnp.float32)
    cnt = jax.lax.dot_general(
        selmask, ones_t, (((1,), (1,)), ((), ())),
        preferred_element_type=jnp.float32)

    accp_ref[...] += psum
    accc_ref[...] += cnt

    @pl.when(i == NUM_TILES - 1)
    def _finalize():
        loss_ref[...] = (N_EXP / (N_TOK * N_TOK)) * jnp.sum(
            accp_ref[...] * accc_ref[...], axis=0, keepdims=True)


def kernel(x, W1, b1, W2, b2):
    b1r = b1.reshape(1, D_HID)
    b2r = b2.reshape(N_EXP, 1)
    idx_t, gates_t, loss = pl.pallas_call(
        _gating_body,
        grid=(NUM_TILES,),
        in_specs=[
            pl.BlockSpec((TILE, D_IN), lambda i: (i, 0)),
            pl.BlockSpec((D_IN, D_HID), lambda i: (0, 0)),
            pl.BlockSpec((1, D_HID), lambda i: (0, 0)),
            pl.BlockSpec((D_HID, N_EXP), lambda i: (0, 0)),
            pl.BlockSpec((N_EXP, 1), lambda i: (0, 0)),
        ],
        out_specs=[
            pl.BlockSpec((TOP_K, TILE), lambda i: (0, i)),
            pl.BlockSpec((TOP_K, TILE), lambda i: (0, i)),
            pl.BlockSpec((1, 1), lambda i: (0, 0)),
        ],
        out_shape=[
            jax.ShapeDtypeStruct((TOP_K, N_TOK), jnp.int32),
            jax.ShapeDtypeStruct((TOP_K, N_TOK), jnp.float32),
            jax.ShapeDtypeStruct((1, 1), jnp.float32),
        ],
        scratch_shapes=[pltpu.VMEM((N_EXP, 1), jnp.float32),
                        pltpu.VMEM((N_EXP, 1), jnp.float32)],
    )(x, W1, b1r, W2, b2r)
    return idx_t.T, gates_t.T, loss.reshape(())
